# baseline (speedup 1.0000x reference)
import functools

import jax
import jax.numpy as jnp
from jax.experimental import pallas as pl
from jax.experimental.pallas import tpu as pltpu

HIDDEN = 768
NUM_EXPERTS = 8
TOP_K = 2

CHUNK = 2048
NBUF = 4


def _router_body(x_hbm, wt_ref, logits_ref, idx_ref, w_ref, xbuf, sem):
    i = pl.program_id(0)
    nchunk = pl.num_programs(0)
    slot = jax.lax.rem(i, NBUF)

    def start(c, s):
        pltpu.make_async_copy(
            x_hbm.at[pl.ds(c * CHUNK, CHUNK)], xbuf.at[s], sem.at[s]
        ).start()

    @pl.when(i == 0)
    def _():
        for b in range(NBUF):
            start(b, b)

    pltpu.make_async_copy(
        x_hbm.at[pl.ds(i * CHUNK, CHUNK)], xbuf.at[slot], sem.at[slot]
    ).wait()

    xb = xbuf[slot]
    logits = jnp.dot(xb, wt_ref[...], preferred_element_type=jnp.float32)

    lt = logits.T
    logits_ref[...] = lt
    iota = jax.lax.broadcasted_iota(jnp.int32, lt.shape, 0)
    m1 = jnp.max(lt, axis=0, keepdims=True)
    i1 = jnp.min(jnp.where(lt == m1, iota, NUM_EXPERTS), axis=0,
                 keepdims=True)
    masked = jnp.where(iota == i1, -1e30, lt)
    m2 = jnp.max(masked, axis=0, keepdims=True)
    i2 = jnp.min(jnp.where(masked == m2, iota, NUM_EXPERTS), axis=0,
                 keepdims=True)
    e = jnp.exp(m2 - m1)
    w1 = 1.0 / (1.0 + e)
    w2 = 1.0 - w1
    idx_ref[...] = jnp.concatenate([i1, i2], axis=0)
    w_ref[...] = jnp.concatenate([w1, w2], axis=0)

    @pl.when(i + NBUF < nchunk)
    def _():
        start(i + NBUF, slot)


@jax.jit
def kernel(x, W):
    b, s, h = x.shape
    n = b * s
    x_flat = x.reshape(n, h)
    wt = W.T
    logits_t, idx_t, w_t = pl.pallas_call(
        _router_body,
        grid=(n // CHUNK,),
        in_specs=[
            pl.BlockSpec(memory_space=pltpu.HBM),
            pl.BlockSpec((h, NUM_EXPERTS), lambda i: (0, 0)),
        ],
        out_specs=[
            pl.BlockSpec((NUM_EXPERTS, CHUNK), lambda i: (0, i)),
            pl.BlockSpec((TOP_K, CHUNK), lambda i: (0, i)),
            pl.BlockSpec((TOP_K, CHUNK), lambda i: (0, i)),
        ],
        out_shape=[
            jax.ShapeDtypeStruct((NUM_EXPERTS, n), jnp.float32),
            jax.ShapeDtypeStruct((TOP_K, n), jnp.int32),
            jax.ShapeDtypeStruct((TOP_K, n), jnp.float32),
        ],
        scratch_shapes=[
            pltpu.VMEM((NBUF, CHUNK, HIDDEN), jnp.float32),
            pltpu.SemaphoreType.DMA((NBUF,)),
        ],
    )(x_flat, wt)
    return (logits_t.T, idx_t.T, w_t.T)


# speedup vs baseline: 1.0169x; 1.0169x over previous
import functools

import jax
import jax.numpy as jnp
from jax.experimental import pallas as pl
from jax.experimental.pallas import tpu as pltpu

HIDDEN = 768
NUM_EXPERTS = 8
TOP_K = 2

CHUNK = 1024
NBUF = 8


def _router_body(x_hbm, wt_ref, logits_ref, idx_ref, w_ref, xbuf, sem):
    i = pl.program_id(0)
    nchunk = pl.num_programs(0)
    slot = jax.lax.rem(i, NBUF)

    def start(c, s):
        pltpu.make_async_copy(
            x_hbm.at[pl.ds(c * CHUNK, CHUNK)], xbuf.at[s], sem.at[s]
        ).start()

    @pl.when(i == 0)
    def _():
        for b in range(NBUF):
            start(b, b)

    pltpu.make_async_copy(
        x_hbm.at[pl.ds(i * CHUNK, CHUNK)], xbuf.at[slot], sem.at[slot]
    ).wait()

    xb = xbuf[slot]
    logits = jnp.dot(xb, wt_ref[...], preferred_element_type=jnp.float32)

    lt = logits.T
    logits_ref[...] = lt
    iota = jax.lax.broadcasted_iota(jnp.int32, lt.shape, 0)
    m1 = jnp.max(lt, axis=0, keepdims=True)
    i1 = jnp.min(jnp.where(lt == m1, iota, NUM_EXPERTS), axis=0,
                 keepdims=True)
    masked = jnp.where(iota == i1, -1e30, lt)
    m2 = jnp.max(masked, axis=0, keepdims=True)
    i2 = jnp.min(jnp.where(masked == m2, iota, NUM_EXPERTS), axis=0,
                 keepdims=True)
    e = jnp.exp(m2 - m1)
    w1 = 1.0 / (1.0 + e)
    w2 = 1.0 - w1
    idx_ref[...] = jnp.concatenate([i1, i2], axis=0)
    w_ref[...] = jnp.concatenate([w1, w2], axis=0)

    @pl.when(i + NBUF < nchunk)
    def _():
        start(i + NBUF, slot)


@jax.jit
def kernel(x, W):
    b, s, h = x.shape
    n = b * s
    x_flat = x.reshape(n, h)
    wt = W.T
    logits_t, idx_t, w_t = pl.pallas_call(
        _router_body,
        grid=(n // CHUNK,),
        in_specs=[
            pl.BlockSpec(memory_space=pltpu.HBM),
            pl.BlockSpec((h, NUM_EXPERTS), lambda i: (0, 0)),
        ],
        out_specs=[
            pl.BlockSpec((NUM_EXPERTS, CHUNK), lambda i: (0, i)),
            pl.BlockSpec((TOP_K, CHUNK), lambda i: (0, i)),
            pl.BlockSpec((TOP_K, CHUNK), lambda i: (0, i)),
        ],
        out_shape=[
            jax.ShapeDtypeStruct((NUM_EXPERTS, n), jnp.float32),
            jax.ShapeDtypeStruct((TOP_K, n), jnp.int32),
            jax.ShapeDtypeStruct((TOP_K, n), jnp.float32),
        ],
        scratch_shapes=[
            pltpu.VMEM((NBUF, CHUNK, HIDDEN), jnp.float32),
            pltpu.SemaphoreType.DMA((NBUF,)),
        ],
    )(x_flat, wt)
    return (logits_t.T, idx_t.T, w_t.T)
